# baseline (speedup 1.0000x reference)
import functools

import jax
import jax.numpy as jnp
from jax import lax
from jax.experimental import pallas as pl
from jax.experimental.pallas import tpu as pltpu
from jax.experimental.pallas import tpu_sc as plsc

B, S, D = 16, 2048, 1024
DH = D // 2
K = 64
NV = DH // 16
NT = 16
RING = 3
AHEAD = RING - 1
NCHMAX = B * 8

_mesh = plsc.VectorSubcoreMesh(core_axis_name="c", subcore_axis_name="s")


@functools.partial(
    pl.kernel,
    out_type=jax.ShapeDtypeStruct((B, D), jnp.float32),
    mesh=_mesh,
    scratch_types=[
        pltpu.VMEM((RING, K, DH), jnp.float32),
        pltpu.VMEM((B * DH,), jnp.float32),
        pltpu.VMEM((32,), jnp.int32),
        pltpu.VMEM((NT, DH), jnp.float32),
        pltpu.VMEM((DH,), jnp.float32),
        pltpu.VMEM_SHARED((NT, B * DH), jnp.float32),
        pltpu.SMEM((4, NCHMAX), jnp.int32),
        pltpu.SemaphoreType.DMA,
        pltpu.SemaphoreType.DMA,
        pltpu.SemaphoreType.DMA,
    ],
)
def _sc_mean(x_hbm, len_hbm, out_hbm, buf, acc, lenv, redbuf, outb, shared,
             desc, *sems):
    c = lax.axis_index("c")
    s = lax.axis_index("s")
    dh0 = c * DH
    pltpu.sync_copy(len_hbm, lenv.at[pl.ds(0, 16)])
    zero = jnp.zeros((16,), jnp.float32)

    def build_b(b, g):
        len_b = lenv[pl.ds(b, 16)][0]
        q = ((len_b + NT * K - 1) // (NT * K)) * K
        sl = lax.rem(s + b, NT)
        start = sl * q
        cnt = jnp.clip(len_b - start, 0, q)
        nch = (cnt + K - 1) // K

        def build_j(j, g2):
            raw = start + j * K
            dstart = jnp.minimum(raw, S - K)
            d = raw - dstart
            rmax = jnp.minimum(K, cnt - j * K)
            desc[0, g2] = b
            desc[1, g2] = dstart
            desc[2, g2] = d
            desc[3, g2] = d + rmax
            return g2 + 1

        return lax.fori_loop(0, nch, build_j, g)

    nch_tot = lax.fori_loop(0, B, build_b, jnp.int32(0))

    def zero_b(b, carry):
        for v in range(NV):
            acc[pl.ds(b * DH + v * 16, 16)] = zero
        return carry

    lax.fori_loop(0, B, zero_b, jnp.int32(0))

    def issue(g):
        dstart = pl.multiple_of(desc[1, g], 8)
        src = x_hbm.at[desc[0, g], pl.ds(dstart, K), pl.ds(dh0, DH)]
        slot = lax.rem(g, RING)
        for r in range(RING):
            @pl.when(slot == r)
            def _(r=r):
                pltpu.async_copy(src, buf.at[r], sems[r])

    def wait_slot(g):
        slot = lax.rem(g, RING)
        for r in range(RING):
            @pl.when(slot == r)
            def _(r=r):
                pltpu.make_async_copy(
                    x_hbm.at[0, pl.ds(0, K), pl.ds(0, DH)],
                    buf.at[r], sems[r]).wait()

    for gp in range(AHEAD):
        @pl.when(gp < nch_tot)
        def _(gp=gp):
            issue(jnp.int32(gp))

    def g_body(g, carry):
        bprev = carry[0]
        accs = carry[1:]
        b = desc[0, g]
        d0 = desc[2, g]
        d1 = desc[3, g]

        @pl.when(b != bprev)
        def _flush():
            for v in range(NV):
                acc[pl.ds(bprev * DH + v * 16, 16)] = accs[v]

        keep = (b == bprev).astype(jnp.float32)
        accs = tuple(a * keep for a in accs)

        wait_slot(g)

        @pl.when(g + AHEAD < nch_tot)
        def _issue_next():
            issue(g + AHEAD)

        slot = lax.rem(g, RING)

        def row(r, a):
            return tuple(a[v] + buf[slot, r, pl.ds(v * 16, 16)]
                         for v in range(NV))

        def row4(i4, a):
            base = d0 + i4 * 4
            for u in range(4):
                a = tuple(a[v] + buf[slot, base + u, pl.ds(v * 16, 16)]
                          for v in range(NV))
            return a

        nfull4 = (d1 - d0) // 4
        accs = lax.fori_loop(0, nfull4, row4, accs)
        accs = lax.fori_loop(d0 + nfull4 * 4, d1, row, accs)
        return (b,) + accs

    first_b = desc[0, 0]
    final = lax.fori_loop(0, nch_tot, g_body, (first_b,) + (zero,) * NV)

    @pl.when(nch_tot > 0)
    def _final_flush():
        blast = final[0]
        for v in range(NV):
            acc[pl.ds(blast * DH + v * 16, 16)] = final[1 + v]

    pltpu.sync_copy(acc, shared.at[s])
    plsc.subcore_barrier()
    pltpu.sync_copy(shared.at[pl.ds(0, NT), pl.ds(s * DH, DH)], redbuf)

    def red(t, a):
        return tuple(a[v] + redbuf[t, pl.ds(v * 16, 16)]
                     for v in range(NV))

    tot = lax.fori_loop(0, NT, red, (zero,) * NV)
    len_s = lenv[pl.ds(s, 16)][0]
    inv_v = jnp.full((16,), 1.0, jnp.float32) / len_s.astype(jnp.float32)
    for v in range(NV):
        outb[pl.ds(v * 16, 16)] = tot[v] * inv_v
    pltpu.sync_copy(outb, out_hbm.at[s, pl.ds(dh0, DH)])


def kernel(input, data_mask, length):
    del data_mask
    return _sc_mean(input, length.astype(jnp.int32))


# speedup vs baseline: 1.3647x; 1.3647x over previous
import functools

import jax
import jax.numpy as jnp
from jax import lax
from jax.experimental import pallas as pl
from jax.experimental.pallas import tpu as pltpu
from jax.experimental.pallas import tpu_sc as plsc

B, S, D = 16, 2048, 1024
DH = D // 2
K = 64
NV = DH // 16
NT = 16
RING = 3
AHEAD = RING - 1
NCHMAX = B * 8

_mesh = plsc.VectorSubcoreMesh(core_axis_name="c", subcore_axis_name="s")


@functools.partial(
    pl.kernel,
    out_type=jax.ShapeDtypeStruct((B, D), jnp.float32),
    mesh=_mesh,
    scratch_types=[
        pltpu.VMEM((RING, K, DH), jnp.float32),
        pltpu.VMEM((B * DH,), jnp.float32),
        pltpu.VMEM((32,), jnp.int32),
        pltpu.VMEM((NT, DH), jnp.float32),
        pltpu.VMEM((DH,), jnp.float32),
        pltpu.VMEM_SHARED((NT, B * DH), jnp.float32),
        pltpu.SMEM((4, NCHMAX), jnp.int32),
        pltpu.SemaphoreType.DMA,
        pltpu.SemaphoreType.DMA,
        pltpu.SemaphoreType.DMA,
    ],
)
def _sc_mean(x_hbm, len_hbm, out_hbm, buf, acc, lenv, redbuf, outb, shared,
             desc, *sems):
    c = lax.axis_index("c")
    s = lax.axis_index("s")
    dh0 = c * DH
    pltpu.sync_copy(len_hbm, lenv.at[pl.ds(0, 16)])
    zero = jnp.zeros((16,), jnp.float32)

    def build_b(b, g):
        len_b = lenv[pl.ds(b, 16)][0]
        q = ((len_b + NT * K - 1) // (NT * K)) * K
        sl = lax.rem(s + b, NT)
        start = sl * q
        cnt = jnp.clip(len_b - start, 0, q)
        nch = (cnt + K - 1) // K

        def build_j(j, g2):
            raw = start + j * K
            dstart = jnp.minimum(raw, S - K)
            d = raw - dstart
            rmax = jnp.minimum(K, cnt - j * K)
            desc[0, g2] = b
            desc[1, g2] = dstart
            desc[2, g2] = d
            desc[3, g2] = d + rmax
            return g2 + 1

        return lax.fori_loop(0, nch, build_j, g)

    nch_tot = lax.fori_loop(0, B, build_b, jnp.int32(0))

    def zero_b(b, carry):
        for v in range(NV):
            acc[pl.ds(b * DH + v * 16, 16)] = zero
        return carry

    lax.fori_loop(0, B, zero_b, jnp.int32(0))

    def issue(g):
        dstart = pl.multiple_of(desc[1, g], 8)
        src = x_hbm.at[desc[0, g], pl.ds(dstart, K), pl.ds(dh0, DH)]
        slot = lax.rem(g, RING)
        for r in range(RING):
            @pl.when(slot == r)
            def _(r=r):
                pltpu.async_copy(src, buf.at[r], sems[r])

    def wait_slot(g):
        slot = lax.rem(g, RING)
        for r in range(RING):
            @pl.when(slot == r)
            def _(r=r):
                pltpu.make_async_copy(
                    x_hbm.at[0, pl.ds(0, K), pl.ds(0, DH)],
                    buf.at[r], sems[r]).wait()

    for gp in range(AHEAD):
        @pl.when(gp < nch_tot)
        def _(gp=gp):
            issue(jnp.int32(gp))

    def g_body(g, carry):
        bprev = carry[0]
        accs = carry[1:]
        b = desc[0, g]
        d0 = desc[2, g]
        d1 = desc[3, g]

        @pl.when(b != bprev)
        def _flush():
            for v in range(NV):
                acc[pl.ds(bprev * DH + v * 16, 16)] = accs[v]

        keep = (b == bprev).astype(jnp.float32)
        accs = tuple(a * keep for a in accs)

        wait_slot(g)

        @pl.when(g + AHEAD < nch_tot)
        def _issue_next():
            issue(g + AHEAD)

        slot = lax.rem(g, RING)

        def row(r, a):
            return tuple(a[v] + buf[slot, r, pl.ds(v * 16, 16)]
                         for v in range(NV))

        accs = lax.fori_loop(d0, d1, row, accs)
        return (b,) + accs

    first_b = desc[0, 0]
    final = lax.fori_loop(0, nch_tot, g_body, (first_b,) + (zero,) * NV)

    @pl.when(nch_tot > 0)
    def _final_flush():
        blast = final[0]
        for v in range(NV):
            acc[pl.ds(blast * DH + v * 16, 16)] = final[1 + v]

    pltpu.sync_copy(acc, shared.at[s])
    plsc.subcore_barrier()
    pltpu.sync_copy(shared.at[pl.ds(0, NT), pl.ds(s * DH, DH)], redbuf)

    def red(t, a):
        return tuple(a[v] + redbuf[t, pl.ds(v * 16, 16)]
                     for v in range(NV))

    tot = lax.fori_loop(0, NT, red, (zero,) * NV)
    len_s = lenv[pl.ds(s, 16)][0]
    inv_v = jnp.full((16,), 1.0, jnp.float32) / len_s.astype(jnp.float32)
    for v in range(NV):
        outb[pl.ds(v * 16, 16)] = tot[v] * inv_v
    pltpu.sync_copy(outb, out_hbm.at[s, pl.ds(dh0, DH)])


def kernel(input, data_mask, length):
    del data_mask
    return _sc_mean(input, length.astype(jnp.int32))
